# baseline (speedup 1.0000x reference)
import functools

import jax
import jax.numpy as jnp
from jax import lax
from jax.experimental import pallas as pl
from jax.experimental.pallas import tpu as pltpu
from jax.experimental.pallas import tpu_sc as plsc

HIDDEN = 32

NUM_CORES = 2
NUM_SUBCORES = 16
NW = NUM_CORES * NUM_SUBCORES

GRP_B = 32
R_BLK = 512

_mesh = plsc.VectorSubcoreMesh(core_axis_name="c", subcore_axis_name="s")


def _make_repack(vocab: int):
    n_full = vocab // R_BLK
    tail_rows = vocab - n_full * R_BLK
    packed = vocab * HIDDEN // 128
    n_iter = (n_full + NW - 1) // NW

    @functools.partial(
        pl.kernel,
        mesh=_mesh,
        out_type=jax.ShapeDtypeStruct((packed, 128), jnp.float32),
        scratch_types=[
            pltpu.VMEM((4, HIDDEN, 128), jnp.float32),
            pltpu.VMEM((128, 128), jnp.float32),
        ],
        compiler_params=pltpu.CompilerParams(needs_layout_passes=False),
    )
    def repack_kernel(src_hbm, tail_hbm, out_hbm, in_v, out_v):
        wid = lax.axis_index("s") * NUM_CORES + lax.axis_index("c")
        iota = lax.iota(jnp.int32, 16)
        d_lo = iota
        d_hi = iota + 16

        def transpose_block(n_pr):
            def prbody(pr, c):
                for cg in range(8):
                    rloc = pr * 4 + (cg // 2)
                    chunk = lax.shift_right_logical(rloc, 7)
                    rl = lax.bitwise_and(rloc, 127)
                    d_vec = d_hi if (cg % 2) else d_lo
                    rl_vec = jnp.full((16,), 0, jnp.int32) + rl
                    val = plsc.load_gather(in_v.at[chunk], [d_vec, rl_vec])
                    out_v[pr, pl.ds(cg * 16, 16)] = val
                return c
            lax.fori_loop(0, n_pr, prbody, 0)

        def block(i, carry):
            bid = i * NW + wid

            @pl.when(bid < n_full)
            def _do():
                r0 = pl.multiple_of(bid * R_BLK, R_BLK)
                for ch in range(4):
                    pltpu.sync_copy(
                        src_hbm.at[:, pl.ds(r0 + ch * 128, 128)],
                        in_v.at[ch])
                transpose_block(128)
                pltpu.sync_copy(
                    out_v, out_hbm.at[pl.ds(bid * 128, 128)])
            return carry

        lax.fori_loop(0, n_iter, block, 0)

        if tail_rows:
            @pl.when(wid == 0)
            def _tail():
                pltpu.sync_copy(tail_hbm, in_v.at[0])
                transpose_block(tail_rows // 4)
                pltpu.sync_copy(
                    out_v.at[pl.ds(0, tail_rows // 4)],
                    out_hbm.at[pl.ds(n_full * 128, tail_rows // 4)])

    return repack_kernel


def _make_gather(batch: int, hist: int):
    b_per_w = batch // NW
    n_grp = b_per_w // GRP_B
    n_pairs = n_grp // 2

    @functools.partial(
        pl.kernel,
        mesh=_mesh,
        out_type=jax.ShapeDtypeStruct((batch, hist, HIDDEN), jnp.float32),
        scratch_types=[
            pltpu.VMEM((2, GRP_B, hist), jnp.int32),
            pltpu.VMEM((2, GRP_B, hist, HIDDEN), jnp.float32),
            pltpu.SemaphoreType.DMA,
            pltpu.SemaphoreType.DMA,
            pltpu.SemaphoreType.DMA,
            pltpu.SemaphoreType.DMA,
            pltpu.SemaphoreType.DMA,
        ],
        compiler_params=pltpu.CompilerParams(use_tc_tiling_on_sc=False),
    )
    def gather_kernel(idx_hbm, table_hbm, out_hbm, idx_v, rows_v,
                      sem_i0, sem_i1, sem_o0, sem_o1, sem_g):
        sem_idx = (sem_i0, sem_i1)
        sem_out = (sem_o0, sem_o1)
        wid = lax.axis_index("s") * NUM_CORES + lax.axis_index("c")
        b_base = wid * b_per_w

        def start_idx(g, b):
            pltpu.async_copy(
                idx_hbm.at[pl.ds(b_base + g * GRP_B, GRP_B)],
                idx_v.at[b], sem_idx[b],
            )

        def wait_idx(b):
            pltpu.make_async_copy(
                idx_hbm.at[pl.ds(b_base, GRP_B)],
                idx_v.at[b], sem_idx[b],
            ).wait()

        def run_gathers(b):
            copies = []
            for j in range(GRP_B):
                copies.append(
                    pltpu.async_copy(
                        table_hbm.at[idx_v.at[b].at[j]],
                        rows_v.at[b].at[j],
                        sem_g,
                    )
                )
            for c in copies:
                c.wait()

        def start_out(g, b):
            pltpu.async_copy(
                rows_v.at[b],
                out_hbm.at[pl.ds(b_base + g * GRP_B, GRP_B)],
                sem_out[b],
            )

        def wait_out(b):
            pltpu.make_async_copy(
                rows_v.at[b],
                out_hbm.at[pl.ds(b_base, GRP_B)], sem_out[b],
            ).wait()

        start_idx(0, 0)
        start_idx(1, 1)

        def pair_body(p, carry):
            for b in range(2):
                g = 2 * p + b
                wait_idx(b)

                @pl.when(g >= 2)
                def _wait_out():
                    wait_out(b)

                run_gathers(b)

                @pl.when(g + 2 < n_grp)
                def _prefetch_idx():
                    start_idx(g + 2, b)

                start_out(g, b)
            return carry

        lax.fori_loop(0, n_pairs, pair_body, 0)

        wait_out(0)
        wait_out(1)

    return gather_kernel


def kernel(indices, table):
    batch, hist = indices.shape
    vocab = table.shape[0]
    n_full = vocab // R_BLK
    fb_rows = n_full * R_BLK
    tail_rows = vocab - fb_rows

    table_t = jnp.swapaxes(table, 0, 1)
    tail = jnp.swapaxes(
        jnp.pad(table[fb_rows:], ((0, 128 - tail_rows), (0, 0))), 0, 1)
    packed = _make_repack(vocab)(table_t, tail)
    table_lin = packed.reshape(vocab, HIDDEN)

    out = _make_gather(batch, hist)(indices, table_lin)
    return out


# speedup vs baseline: 1.4991x; 1.4991x over previous
import functools

import jax
import jax.numpy as jnp
from jax import lax
from jax.experimental import pallas as pl
from jax.experimental.pallas import tpu as pltpu
from jax.experimental.pallas import tpu_sc as plsc

HIDDEN = 32

NUM_CORES = 2
NUM_SUBCORES = 16
NW = NUM_CORES * NUM_SUBCORES

GRP_B = 32


def _make_gather(batch: int, hist: int):
    b_per_w = batch // NW
    n_grp = b_per_w // GRP_B
    n_pairs = n_grp // 2
    mesh = plsc.VectorSubcoreMesh(core_axis_name="c", subcore_axis_name="s")

    @functools.partial(
        pl.kernel,
        mesh=mesh,
        out_type=jax.ShapeDtypeStruct((batch, hist, HIDDEN), jnp.float32),
        scratch_types=[
            pltpu.VMEM((2, GRP_B, hist), jnp.int32),
            pltpu.VMEM((2, GRP_B, hist, HIDDEN), jnp.float32),
            pltpu.SemaphoreType.DMA,
            pltpu.SemaphoreType.DMA,
            pltpu.SemaphoreType.DMA,
            pltpu.SemaphoreType.DMA,
            pltpu.SemaphoreType.DMA,
        ],
        compiler_params=pltpu.CompilerParams(use_tc_tiling_on_sc=False),
    )
    def gather_kernel(idx_hbm, table_hbm, out_hbm, idx_v, rows_v,
                      sem_i0, sem_i1, sem_o0, sem_o1, sem_g):
        sem_idx = (sem_i0, sem_i1)
        sem_out = (sem_o0, sem_o1)
        wid = lax.axis_index("s") * NUM_CORES + lax.axis_index("c")
        b_base = wid * b_per_w

        def start_idx(g, b):
            pltpu.async_copy(
                idx_hbm.at[pl.ds(b_base + g * GRP_B, GRP_B)],
                idx_v.at[b], sem_idx[b],
            )

        def wait_idx(b):
            pltpu.make_async_copy(
                idx_hbm.at[pl.ds(b_base, GRP_B)],
                idx_v.at[b], sem_idx[b],
            ).wait()

        def run_gathers(b):
            copies = []
            for j in range(GRP_B):
                copies.append(
                    pltpu.async_copy(
                        table_hbm.at[idx_v.at[b].at[j]],
                        rows_v.at[b].at[j],
                        sem_g,
                    )
                )
            for c in copies:
                c.wait()

        def start_out(g, b):
            pltpu.async_copy(
                rows_v.at[b],
                out_hbm.at[pl.ds(b_base + g * GRP_B, GRP_B)],
                sem_out[b],
            )

        def wait_out(b):
            pltpu.make_async_copy(
                rows_v.at[b],
                out_hbm.at[pl.ds(b_base, GRP_B)], sem_out[b],
            ).wait()

        start_idx(0, 0)
        start_idx(1, 1)

        def pair_body(p, carry):
            for b in range(2):
                g = 2 * p + b
                wait_idx(b)

                @pl.when(g >= 2)
                def _wait_out():
                    wait_out(b)

                run_gathers(b)

                @pl.when(g + 2 < n_grp)
                def _prefetch_idx():
                    start_idx(g + 2, b)

                start_out(g, b)
            return carry

        lax.fori_loop(0, n_pairs, pair_body, 0)

        wait_out(0)
        wait_out(1)

    return gather_kernel


def kernel(indices, table):
    batch, hist = indices.shape
    return _make_gather(batch, hist)(indices, table)
